# baseline (speedup 1.0000x reference)
import functools

import jax
import jax.numpy as jnp
from jax import lax
from jax.experimental import pallas as pl
from jax.experimental.pallas import tpu as pltpu
from jax.experimental.pallas import tpu_sc as plsc

DIM = 64
BT = 128
NC = 2
NS = 16
NW = NC * NS


@functools.lru_cache(maxsize=None)
def _make_sc_gather(b: int, l: int, vocab: int):
    K = next(k for k in (5, 4, 2, 1) if l % (2 * k) == 0)
    assert b == BT * NW
    n_grp = l // K
    rpl = b // 2
    mesh = plsc.VectorSubcoreMesh(core_axis_name="c", subcore_axis_name="s")

    @functools.partial(
        pl.kernel,
        mesh=mesh,
        out_type=jax.ShapeDtypeStruct((l * rpl, 2 * DIM), jnp.float32),
        scratch_types=[
            pltpu.VMEM((l, BT), jnp.int32),
            pltpu.VMEM((2, K, BT, DIM), jnp.float32),
            pltpu.SemaphoreType.DMA,
            pltpu.SemaphoreType.DMA,
            pltpu.SemaphoreType.DMA,
            pltpu.SemaphoreType.DMA,
        ],
        compiler_params=pltpu.CompilerParams(use_tc_tiling_on_sc=False),
    )
    def sc_gather(idxt_hbm, table_hbm, out_hbm, idx_t, rows,
                  gsem0, gsem1, ssem0, ssem1):
        gsem = (gsem0, gsem1)
        ssem = (ssem0, ssem1)
        wid = lax.axis_index("s") * NC + lax.axis_index("c")
        grp = wid // 2
        half = wid % 2
        gbase = grp * 2 * BT + half * (BT // 2)
        pltpu.sync_copy(idxt_hbm.at[:, pl.ds(gbase, BT // 2)],
                        idx_t.at[:, pl.ds(0, BT // 2)])
        pltpu.sync_copy(idxt_hbm.at[:, pl.ds(gbase + BT, BT // 2)],
                        idx_t.at[:, pl.ds(BT // 2, BT // 2)])

        def fire_gathers(r, g):
            for j in range(K):
                pltpu.async_copy(
                    table_hbm.at[idx_t.at[g * K + j]], rows.at[r, j],
                    gsem[r])

        def drain_gathers(r):
            pltpu.make_async_copy(
                table_hbm.at[pl.ds(0, K * BT)],
                rows.at[r], gsem[r]).wait()

        def fire_scatters(r, g):
            for j in range(K):
                row = (g * K + j) * rpl + grp * BT + half * (BT // 2)
                pltpu.async_copy(
                    rows.at[r, j, pl.ds(0, BT // 2), :],
                    out_hbm.at[pl.ds(row, BT // 2), pl.ds(0, DIM)],
                    ssem[r])
                pltpu.async_copy(
                    rows.at[r, j, pl.ds(BT // 2, BT // 2), :],
                    out_hbm.at[pl.ds(row, BT // 2), pl.ds(DIM, DIM)],
                    ssem[r])

        def drain_scatters(r):
            for j in range(2 * K):
                pltpu.make_async_copy(
                    rows.at[r, 0, pl.ds(0, BT // 2), :],
                    out_hbm.at[pl.ds(0, BT // 2), pl.ds(0, DIM)],
                    ssem[r]).wait()

        fire_gathers(0, 0)
        fire_gathers(1, 1)

        @pl.loop(0, n_grp, step=2)
        def _(g):
            drain_gathers(0)
            fire_scatters(0, g)
            drain_gathers(1)
            fire_scatters(1, g + 1)
            drain_scatters(0)

            @pl.when(g + 2 < n_grp)
            def _():
                fire_gathers(0, g + 2)

            drain_scatters(1)

            @pl.when(g + 3 < n_grp)
            def _():
                fire_gathers(1, g + 3)

    return sc_gather


@functools.lru_cache(maxsize=None)
def _make_tc_transpose(b: int, l: int):
    nbg = b // (2 * BT)
    nc = l // 8

    def process(y_ref, o_ref):
        for s2 in range(4):
            for g in range(nbg):
                blk = y_ref[pl.ds((s2 * nbg + g) * BT, BT), :]
                left = jnp.transpose(blk[:, :DIM])
                right = jnp.transpose(blk[:, DIM:])
                o_ref[s2, :, pl.ds(g * 2 * BT, 2 * BT)] = jnp.concatenate(
                    [left, right], axis=1)

    def tc_transpose_body(y1_ref, y2_ref, o_ref):
        s = pl.program_id(0)

        @pl.when(s < nc)
        def _():
            process(y1_ref, o_ref)

        @pl.when(s >= nc)
        def _():
            process(y2_ref, o_ref)

    return pl.pallas_call(
        tc_transpose_body,
        grid=(l // 4,),
        in_specs=[
            pl.BlockSpec((2 * b, 2 * DIM),
                         lambda s: (jnp.minimum(s, nc - 1), 0)),
            pl.BlockSpec((2 * b, 2 * DIM),
                         lambda s: (jnp.maximum(s - nc, 0), 0)),
        ],
        out_specs=pl.BlockSpec((4, DIM, b), lambda s: (s, 0, 0)),
        out_shape=jax.ShapeDtypeStruct((l, DIM, b), jnp.float32),
    )


def kernel(x, W):
    b, l = x.shape
    lc = l // 2
    idxt = x.T
    sc = _make_sc_gather(b, lc, W.shape[0])
    y1 = sc(idxt[:lc], W)
    y2 = sc(idxt[lc:], W)
    x3 = _make_tc_transpose(b, l)(y1, y2)
    return x3.transpose(2, 0, 1)


# speedup vs baseline: 1.0722x; 1.0722x over previous
import functools

import jax
import jax.numpy as jnp
from jax import lax
from jax.experimental import pallas as pl
from jax.experimental.pallas import tpu as pltpu
from jax.experimental.pallas import tpu_sc as plsc

DIM = 64
BT = 128
NC = 2
NS = 16
NW = NC * NS


@functools.lru_cache(maxsize=None)
def _make_sc_gather(b: int, l: int, vocab: int):
    K = next(k for k in (5, 4, 2, 1) if l % (2 * k) == 0)
    assert b == BT * NW
    n_grp = l // K
    rpl = b // 2
    mesh = plsc.VectorSubcoreMesh(core_axis_name="c", subcore_axis_name="s")

    @functools.partial(
        pl.kernel,
        mesh=mesh,
        out_type=jax.ShapeDtypeStruct((l * rpl, 2 * DIM), jnp.float32),
        scratch_types=[
            pltpu.VMEM((l, BT), jnp.int32),
            pltpu.VMEM((2, K, BT, DIM), jnp.float32),
            pltpu.SemaphoreType.DMA,
            pltpu.SemaphoreType.DMA,
            pltpu.SemaphoreType.DMA,
            pltpu.SemaphoreType.DMA,
        ],
        compiler_params=pltpu.CompilerParams(use_tc_tiling_on_sc=False),
    )
    def sc_gather(idxt_hbm, table_hbm, out_hbm, idx_t, rows,
                  gsem0, gsem1, ssem0, ssem1):
        gsem = (gsem0, gsem1)
        ssem = (ssem0, ssem1)
        wid = lax.axis_index("s") * NC + lax.axis_index("c")
        grp = wid // 2
        half = wid % 2
        gbase = grp * 2 * BT + half * (BT // 2)
        pltpu.sync_copy(idxt_hbm.at[:, pl.ds(gbase, BT // 2)],
                        idx_t.at[:, pl.ds(0, BT // 2)])
        pltpu.sync_copy(idxt_hbm.at[:, pl.ds(gbase + BT, BT // 2)],
                        idx_t.at[:, pl.ds(BT // 2, BT // 2)])

        def fire_gathers(r, g):
            for j in range(K):
                pltpu.async_copy(
                    table_hbm.at[idx_t.at[g * K + j]], rows.at[r, j],
                    gsem[r])

        def drain_gathers(r):
            pltpu.make_async_copy(
                table_hbm.at[pl.ds(0, K * BT)],
                rows.at[r], gsem[r]).wait()

        def fire_scatters(r, g):
            for j in range(K):
                row = (g * K + j) * rpl + grp * BT + half * (BT // 2)
                pltpu.async_copy(
                    rows.at[r, j, pl.ds(0, BT // 2), :],
                    out_hbm.at[pl.ds(row, BT // 2), pl.ds(0, DIM)],
                    ssem[r])
                pltpu.async_copy(
                    rows.at[r, j, pl.ds(BT // 2, BT // 2), :],
                    out_hbm.at[pl.ds(row, BT // 2), pl.ds(DIM, DIM)],
                    ssem[r])

        def drain_scatters(r):
            for j in range(2 * K):
                pltpu.make_async_copy(
                    rows.at[r, 0, pl.ds(0, BT // 2), :],
                    out_hbm.at[pl.ds(0, BT // 2), pl.ds(0, DIM)],
                    ssem[r]).wait()

        fire_gathers(0, 0)
        fire_gathers(1, 1)

        @pl.loop(0, n_grp, step=2)
        def _(g):
            drain_gathers(0)
            fire_scatters(0, g)
            drain_gathers(1)
            fire_scatters(1, g + 1)
            drain_scatters(0)

            @pl.when(g + 2 < n_grp)
            def _():
                fire_gathers(0, g + 2)

            drain_scatters(1)

            @pl.when(g + 3 < n_grp)
            def _():
                fire_gathers(1, g + 3)

    return sc_gather


@functools.lru_cache(maxsize=None)
def _make_tc_transpose(b: int, l: int, lc: int, chunk: int):
    nbg = b // (2 * BT)
    nc = lc // 4
    off = chunk * nc

    def process(y_ref, o_ref):
        for s2 in range(4):
            for g in range(nbg):
                blk = y_ref[pl.ds((s2 * nbg + g) * BT, BT), :]
                left = jnp.transpose(blk[:, :DIM])
                right = jnp.transpose(blk[:, DIM:])
                o_ref[s2, :, pl.ds(g * 2 * BT, 2 * BT)] = jnp.concatenate(
                    [left, right], axis=1)

    if chunk == 0:
        def body(y_ref, o_ref):
            process(y_ref, o_ref)

        return pl.pallas_call(
            body,
            grid=(nc,),
            in_specs=[pl.BlockSpec((2 * b, 2 * DIM), lambda s: (s, 0))],
            out_specs=pl.BlockSpec((4, DIM, b), lambda s: (s, 0, 0)),
            out_shape=jax.ShapeDtypeStruct((l, DIM, b), jnp.float32),
        )

    def body(y_ref, prev_ref, o_ref):
        process(y_ref, o_ref)

    return pl.pallas_call(
        body,
        grid=(nc,),
        in_specs=[
            pl.BlockSpec((2 * b, 2 * DIM), lambda s: (s, 0)),
            pl.BlockSpec((4, DIM, b), lambda s: (0, 0, 0)),
        ],
        out_specs=pl.BlockSpec((4, DIM, b), lambda s: (s + off, 0, 0)),
        out_shape=jax.ShapeDtypeStruct((l, DIM, b), jnp.float32),
        input_output_aliases={1: 0},
    )


def kernel(x, W):
    b, l = x.shape
    lc = l // 2
    idxt = x.T
    sc = _make_sc_gather(b, lc, W.shape[0])
    y1 = sc(idxt[:lc], W)
    y2 = sc(idxt[lc:], W)
    x3a = _make_tc_transpose(b, l, lc, 0)(y1)
    x3 = _make_tc_transpose(b, l, lc, 1)(y2, x3a)
    return x3.transpose(2, 0, 1)


# speedup vs baseline: 1.0931x; 1.0195x over previous
import functools

import jax
import jax.numpy as jnp
from jax import lax
from jax.experimental import pallas as pl
from jax.experimental.pallas import tpu as pltpu
from jax.experimental.pallas import tpu_sc as plsc

DIM = 64
BT = 128
NC = 2
NS = 16
NW = NC * NS


@functools.lru_cache(maxsize=None)
def _make_sc_gather(b: int, l: int, vocab: int):
    K = next(k for k in (5, 4, 2, 1) if l % (2 * k) == 0)
    assert b == BT * NW
    n_grp = l // K
    rpl = b // 2
    mesh = plsc.VectorSubcoreMesh(core_axis_name="c", subcore_axis_name="s")

    @functools.partial(
        pl.kernel,
        mesh=mesh,
        out_type=jax.ShapeDtypeStruct((l * rpl, 2 * DIM), jnp.float32),
        scratch_types=[
            pltpu.VMEM((l, BT), jnp.int32),
            pltpu.VMEM((2, K, BT, DIM), jnp.float32),
            pltpu.SemaphoreType.DMA,
            pltpu.SemaphoreType.DMA,
            pltpu.SemaphoreType.DMA,
            pltpu.SemaphoreType.DMA,
        ],
        compiler_params=pltpu.CompilerParams(use_tc_tiling_on_sc=False),
    )
    def sc_gather(idxt_hbm, table_hbm, out_hbm, idx_t, rows,
                  gsem0, gsem1, ssem0, ssem1):
        gsem = (gsem0, gsem1)
        ssem = (ssem0, ssem1)
        wid = lax.axis_index("s") * NC + lax.axis_index("c")
        grp = wid // 2
        half = wid % 2
        gbase = grp * 2 * BT + half * (BT // 2)
        pltpu.sync_copy(idxt_hbm.at[:, pl.ds(gbase, BT // 2)],
                        idx_t.at[:, pl.ds(0, BT // 2)])
        pltpu.sync_copy(idxt_hbm.at[:, pl.ds(gbase + BT, BT // 2)],
                        idx_t.at[:, pl.ds(BT // 2, BT // 2)])

        def fire_gathers(r, g):
            for j in range(K):
                pltpu.async_copy(
                    table_hbm.at[idx_t.at[g * K + j]], rows.at[r, j],
                    gsem[r])

        def drain_gathers(r):
            pltpu.make_async_copy(
                table_hbm.at[pl.ds(0, K * BT)],
                rows.at[r], gsem[r]).wait()

        def fire_scatters(r, g):
            for j in range(K):
                row = (g * K + j) * rpl + grp * BT + half * (BT // 2)
                pltpu.async_copy(
                    rows.at[r, j, pl.ds(0, BT // 2), :],
                    out_hbm.at[pl.ds(row, BT // 2), pl.ds(0, DIM)],
                    ssem[r])
                pltpu.async_copy(
                    rows.at[r, j, pl.ds(BT // 2, BT // 2), :],
                    out_hbm.at[pl.ds(row, BT // 2), pl.ds(DIM, DIM)],
                    ssem[r])

        def drain_scatters(r):
            for j in range(2 * K):
                pltpu.make_async_copy(
                    rows.at[r, 0, pl.ds(0, BT // 2), :],
                    out_hbm.at[pl.ds(0, BT // 2), pl.ds(0, DIM)],
                    ssem[r]).wait()

        fire_gathers(0, 0)
        fire_gathers(1, 1)

        @pl.loop(0, n_grp, step=2)
        def _(g):
            drain_gathers(0)
            fire_scatters(0, g)
            drain_gathers(1)
            fire_scatters(1, g + 1)
            drain_scatters(0)

            @pl.when(g + 2 < n_grp)
            def _():
                fire_gathers(0, g + 2)

            drain_scatters(1)

            @pl.when(g + 3 < n_grp)
            def _():
                fire_gathers(1, g + 3)

    return sc_gather


@functools.lru_cache(maxsize=None)
def _make_tc_transpose(b: int, l: int, lc: int, off_steps: int):
    nbg = b // (2 * BT)
    nc = lc // 4

    def process(y_ref, o_ref):
        for s2 in range(4):
            for g in range(nbg):
                blk = y_ref[pl.ds((s2 * nbg + g) * BT, BT), :]
                left = jnp.transpose(blk[:, :DIM])
                right = jnp.transpose(blk[:, DIM:])
                o_ref[s2, :, pl.ds(g * 2 * BT, 2 * BT)] = jnp.concatenate(
                    [left, right], axis=1)

    if off_steps == 0:
        def body(y_ref, o_ref):
            process(y_ref, o_ref)

        return pl.pallas_call(
            body,
            grid=(nc,),
            in_specs=[pl.BlockSpec((2 * b, 2 * DIM), lambda s: (s, 0))],
            out_specs=pl.BlockSpec((4, DIM, b), lambda s: (s, 0, 0)),
            out_shape=jax.ShapeDtypeStruct((l, DIM, b), jnp.float32),
        )

    def body(y_ref, prev_ref, o_ref):
        process(y_ref, o_ref)

    return pl.pallas_call(
        body,
        grid=(nc,),
        in_specs=[
            pl.BlockSpec((2 * b, 2 * DIM), lambda s: (s, 0)),
            pl.BlockSpec((4, DIM, b), lambda s: (0, 0, 0)),
        ],
        out_specs=pl.BlockSpec((4, DIM, b),
                               lambda s: (s + off_steps, 0, 0)),
        out_shape=jax.ShapeDtypeStruct((l, DIM, b), jnp.float32),
        input_output_aliases={1: 0},
    )


_CHUNKS = (64, 68, 68)


def kernel(x, W):
    b, l = x.shape
    idxt = x.T
    chunks = _CHUNKS if sum(_CHUNKS) == l else (l,)
    ys, s0 = [], 0
    for lc in chunks:
        ys.append(_make_sc_gather(b, lc, W.shape[0])(
            idxt[s0:s0 + lc], W))
        s0 += lc
    x3, steps = None, 0
    for lc, y in zip(chunks, ys):
        tc = _make_tc_transpose(b, l, lc, steps)
        x3 = tc(y) if x3 is None else tc(y, x3)
        steps += lc // 4
    return x3.transpose(2, 0, 1)
